# baseline (speedup 1.0000x reference)
import functools

import jax
import jax.numpy as jnp
from jax import lax
from jax.experimental import pallas as pl
from jax.experimental.pallas import tpu as pltpu
from jax.experimental.pallas import tpu_sc as plsc

_NC, _NS, _L = 2, 16, 16
_NW = _NC * _NS


def _mask_sc_body(w_hbm, p_hbm, out_hbm, w_v, p_v):
    n_per_w = w_v.shape[0]
    wid = lax.axis_index("s") * _NC + lax.axis_index("c")
    base = wid * n_per_w
    pltpu.sync_copy(w_hbm.at[pl.ds(base, n_per_w)], w_v)
    pltpu.sync_copy(p_hbm.at[pl.ds(base, n_per_w)], p_v)

    def body(i, carry):
        sl = pl.ds(i * _L, _L)
        w_v[sl] = jnp.where(p_v[sl] != 0.0, 0.0, w_v[sl])
        return carry

    lax.fori_loop(0, n_per_w // _L, body, 0)
    pltpu.sync_copy(w_v, out_hbm.at[pl.ds(base, n_per_w)])


def _masked_weight_sc(weight, pruned):
    n = weight.size
    n_per_w = n // _NW
    mesh = plsc.VectorSubcoreMesh(core_axis_name="c", subcore_axis_name="s")
    fn = functools.partial(
        pl.kernel,
        mesh=mesh,
        out_type=jax.ShapeDtypeStruct((n,), jnp.float32),
        scratch_types=[
            pltpu.VMEM((n_per_w,), jnp.float32),
            pltpu.VMEM((n_per_w,), jnp.float32),
        ],
    )(_mask_sc_body)
    mw = fn(weight.reshape(n), pruned.reshape(n).astype(jnp.float32))
    return mw.reshape(weight.shape)


def _apply_body(img_ref, mw_ref, out_ref):
    out_ref[...] = img_ref[...] * mw_ref[...][None, :, :]


def kernel(image, weight, pruned):
    B, C, H, W = image.shape
    img = image.reshape(B * C, H, W)
    mw = _masked_weight_sc(weight, pruned)
    K = 8
    out = pl.pallas_call(
        _apply_body,
        grid=(B * C // K,),
        in_specs=[
            pl.BlockSpec((K, H, W), lambda i: (i, 0, 0)),
            pl.BlockSpec((H, W), lambda i: (0, 0)),
        ],
        out_specs=pl.BlockSpec((K, H, W), lambda i: (i, 0, 0)),
        out_shape=jax.ShapeDtypeStruct((B * C, H, W), jnp.float32),
    )(img, mw)
    return out.reshape(1, B, C, H, W)


# speedup vs baseline: 1.0181x; 1.0181x over previous
import functools

import jax
import jax.numpy as jnp
from jax import lax
from jax.experimental import pallas as pl
from jax.experimental.pallas import tpu as pltpu
from jax.experimental.pallas import tpu_sc as plsc

_NC, _NS, _L = 2, 16, 16
_NW = _NC * _NS


def _mask_sc_body(w_hbm, p_hbm, out_hbm, w_v, p_v):
    n_per_w = w_v.shape[0]
    wid = lax.axis_index("s") * _NC + lax.axis_index("c")
    base = wid * n_per_w
    pltpu.sync_copy(w_hbm.at[pl.ds(base, n_per_w)], w_v)
    pltpu.sync_copy(p_hbm.at[pl.ds(base, n_per_w)], p_v)

    _UNROLL = 8

    def body(i, carry):
        for u in range(_UNROLL):
            sl = pl.ds((i * _UNROLL + u) * _L, _L)
            w = w_v[sl]
            w_v[sl] = w - w * p_v[sl]
        return carry

    lax.fori_loop(0, n_per_w // (_L * _UNROLL), body, 0)
    pltpu.sync_copy(w_v, out_hbm.at[pl.ds(base, n_per_w)])


def _masked_weight_sc(weight, pruned):
    n = weight.size
    n_per_w = n // _NW
    mesh = plsc.VectorSubcoreMesh(core_axis_name="c", subcore_axis_name="s")
    fn = functools.partial(
        pl.kernel,
        mesh=mesh,
        out_type=jax.ShapeDtypeStruct((n,), jnp.float32),
        scratch_types=[
            pltpu.VMEM((n_per_w,), jnp.float32),
            pltpu.VMEM((n_per_w,), jnp.float32),
        ],
    )(_mask_sc_body)
    mw = fn(weight.reshape(n), pruned.reshape(n).astype(jnp.float32))
    return mw.reshape(weight.shape)


def _apply_body(img_ref, mw_ref, out_ref):
    out_ref[...] = img_ref[...] * mw_ref[...][None, :, :]


def kernel(image, weight, pruned):
    B, C, H, W = image.shape
    img = image.reshape(B * C, H, W)
    mw = _masked_weight_sc(weight, pruned)
    K = 8
    out = pl.pallas_call(
        _apply_body,
        grid=(B * C // K,),
        in_specs=[
            pl.BlockSpec((K, H, W), lambda i: (i, 0, 0)),
            pl.BlockSpec((H, W), lambda i: (0, 0)),
        ],
        out_specs=pl.BlockSpec((K, H, W), lambda i: (i, 0, 0)),
        out_shape=jax.ShapeDtypeStruct((B * C, H, W), jnp.float32),
    )(img, mw)
    return out.reshape(1, B, C, H, W)


# speedup vs baseline: 1.2864x; 1.2636x over previous
import jax
import jax.numpy as jnp
from jax import lax
from jax.experimental import pallas as pl
from jax.experimental.pallas import tpu as pltpu

_NBUF = 3
_SL = 8


def _stream_body(img_hbm, w_ref, p_ref, out_hbm, mw_ref, ibuf, obuf, isem, osem):
    nch = img_hbm.shape[0] // _SL
    mw_ref[...] = jnp.where(p_ref[...], 0.0, w_ref[...])

    def in_slice(c, b, k):
        return pltpu.make_async_copy(
            img_hbm.at[c * _SL + k], ibuf.at[b, k], isem.at[b, k])

    def out_slice(c, b, k):
        return pltpu.make_async_copy(
            obuf.at[b, k], out_hbm.at[c * _SL + k], osem.at[b, k])

    for b in range(_NBUF):
        for k in range(_SL):
            in_slice(b, b, k).start()

    def step(c, carry):
        b = lax.rem(c, _NBUF)

        @pl.when(c >= _NBUF)
        def _():
            for k in range(_SL):
                out_slice(c - _NBUF, b, k).wait()

        for k in range(_SL):
            in_slice(c, b, k).wait()
            obuf[b, k] = ibuf[b, k] * mw_ref[...]
            out_slice(c, b, k).start()

        @pl.when(c + _NBUF < nch)
        def _():
            for k in range(_SL):
                in_slice(c + _NBUF, b, k).start()

        return carry

    lax.fori_loop(0, nch, step, 0)
    for c in range(nch - _NBUF, nch):
        for k in range(_SL):
            out_slice(c, c % _NBUF, k).wait()


def kernel(image, weight, pruned):
    B, C, H, W = image.shape
    img = image.reshape(B * C, H, W)
    out = pl.pallas_call(
        _stream_body,
        in_specs=[
            pl.BlockSpec(memory_space=pl.ANY),
            pl.BlockSpec(memory_space=pltpu.VMEM),
            pl.BlockSpec(memory_space=pltpu.VMEM),
        ],
        out_specs=pl.BlockSpec(memory_space=pl.ANY),
        out_shape=jax.ShapeDtypeStruct((B * C, H, W), jnp.float32),
        scratch_shapes=[
            pltpu.VMEM((H, W), jnp.float32),
            pltpu.VMEM((_NBUF, _SL, H, W), jnp.float32),
            pltpu.VMEM((_NBUF, _SL, H, W), jnp.float32),
            pltpu.SemaphoreType.DMA((_NBUF, _SL)),
            pltpu.SemaphoreType.DMA((_NBUF, _SL)),
        ],
        compiler_params=pltpu.CompilerParams(
            vmem_limit_bytes=62 * 1024 * 1024,
        ),
    )(img, weight, pruned)
    return out.reshape(1, B, C, H, W)


# speedup vs baseline: 1.3038x; 1.0136x over previous
import jax
import jax.numpy as jnp
from jax.experimental import pallas as pl
from jax.experimental.pallas import tpu as pltpu


def _apply_body(img_ref, w_ref, p_ref, out_ref, mw_ref):
    @pl.when(pl.program_id(0) == 0)
    def _():
        mw_ref[...] = jnp.where(p_ref[...], 0.0, w_ref[...])

    out_ref[...] = img_ref[...] * mw_ref[...][None, :, :]


def kernel(image, weight, pruned):
    B, C, H, W = image.shape
    img = image.reshape(B * C, H, W)
    K = 8
    out = pl.pallas_call(
        _apply_body,
        grid=(B * C // K,),
        in_specs=[
            pl.BlockSpec((K, H, W), lambda i: (i, 0, 0)),
            pl.BlockSpec((H, W), lambda i: (0, 0)),
            pl.BlockSpec((H, W), lambda i: (0, 0)),
        ],
        out_specs=pl.BlockSpec((K, H, W), lambda i: (i, 0, 0)),
        out_shape=jax.ShapeDtypeStruct((B * C, H, W), jnp.float32),
        scratch_shapes=[pltpu.VMEM((H, W), jnp.float32)],
        compiler_params=pltpu.CompilerParams(
            dimension_semantics=("arbitrary",),
            skip_device_barrier=True,
        ),
    )(img, weight, pruned)
    return out.reshape(1, B, C, H, W)
